# Initial kernel scaffold; baseline (speedup 1.0000x reference)
import jax
import jax.numpy as jnp
from jax.experimental import pallas as pl


def kernel(x, edge_index, params):
    raise NotImplementedError("write your pallas kernel here")


# speedup vs baseline: 3.9029x; 3.9029x over previous
import functools

import jax
import jax.numpy as jnp
from jax import lax
from jax.experimental import pallas as pl
from jax.experimental.pallas import tpu as pltpu
from jax.experimental.pallas import tpu_sc as plsc

N = 10000
E = 160000
Np = 10240
BNp = 1024
NBLK = Np // BNp
ROWS_PER_TILE = Np // 16
CHUNK = 128
NCH = E // CHUNK
ITERS = (NCH + 31) // 32
BE = 4000
GEB = E // BE

_f32 = jnp.float32
_i32 = jnp.int32

_MESH = dict(core_axis_name="c", subcore_axis_name="s")


def _wid():
    return lax.axis_index("s") * 2 + lax.axis_index("c")


def _agg_pass(y_flat, src, dst, acc_sh, acc_out, zbuf, rows, sidx, didx,
              gidx, sem, cc, nc, sid, cid, wid):
    for zz in range(ROWS_PER_TILE // CHUNK):
        pltpu.sync_copy(
            zbuf, acc_sh.at[pl.ds(sid * ROWS_PER_TILE + zz * CHUNK, CHUNK), :])
    plsc.subcore_barrier()

    def body(i, carry):
        k = wid + i * 32

        @pl.when(k < NCH)
        def _():
            pltpu.sync_copy(src.at[pl.ds(k * CHUNK, CHUNK)], sidx)
            if nc > 1:
                for j in range(CHUNK // 16):
                    gidx[pl.ds(j * 16, 16)] = sidx[pl.ds(j * 16, 16)] + _i32(cc * Np)
                idxref = gidx
            else:
                idxref = sidx
            pltpu.async_copy(y_flat.at[idxref], rows, sem).wait()
            pltpu.sync_copy(dst.at[pl.ds(k * CHUNK, CHUNK)], didx)
            pltpu.sync_copy(rows, acc_sh.at[didx], add=True)

        return carry

    lax.fori_loop(0, ITERS, body, 0)
    plsc.subcore_barrier()
    for zz in range(ROWS_PER_TILE // CHUNK):
        pltpu.sync_copy(
            acc_sh.at[pl.ds(sid * ROWS_PER_TILE + zz * CHUNK, CHUNK), :], rows)
        pltpu.sync_copy(
            rows,
            acc_out.at[cid, cc, pl.ds(sid * ROWS_PER_TILE + zz * CHUNK, CHUNK), :])
    plsc.subcore_barrier()


def _make_sc_deg():
    @functools.partial(
        pl.kernel,
        mesh=plsc.VectorSubcoreMesh(**_MESH),
        out_type=jax.ShapeDtypeStruct((2, Np, 128), _f32),
        scratch_types=[
            pltpu.VMEM((CHUNK,), _i32),
            pltpu.VMEM((CHUNK, 128), _f32),
            pltpu.VMEM((CHUNK, 128), _f32),
            pltpu.VMEM_SHARED((Np, 128), _f32),
        ],
    )
    def k(dst, z128, ones128, deg_out, didx, onesb, stg, deg_sh):
        cid = lax.axis_index("c")
        sid = lax.axis_index("s")
        wid = _wid()
        pltpu.sync_copy(ones128, onesb)
        pltpu.sync_copy(z128, stg)
        for zz in range(ROWS_PER_TILE // CHUNK):
            pltpu.sync_copy(
                stg, deg_sh.at[pl.ds(sid * ROWS_PER_TILE + zz * CHUNK, CHUNK), :])
        plsc.subcore_barrier()

        def dbody(i, carry):
            k_ = wid + i * 32

            @pl.when(k_ < NCH)
            def _():
                pltpu.sync_copy(dst.at[pl.ds(k_ * CHUNK, CHUNK)], didx)
                pltpu.sync_copy(onesb, deg_sh.at[didx], add=True)

            return carry

        lax.fori_loop(0, ITERS, dbody, 0)
        plsc.subcore_barrier()
        for zz in range(ROWS_PER_TILE // CHUNK):
            pltpu.sync_copy(
                deg_sh.at[pl.ds(sid * ROWS_PER_TILE + zz * CHUNK, CHUNK), :], stg)
            pltpu.sync_copy(
                stg,
                deg_out.at[cid, pl.ds(sid * ROWS_PER_TILE + zz * CHUNK, CHUNK), :])

    return k


def _make_sc_agg(nc):
    @functools.partial(
        pl.kernel,
        mesh=plsc.VectorSubcoreMesh(**_MESH),
        out_type=jax.ShapeDtypeStruct((2, nc, Np, 128), _f32),
        scratch_types=[
            pltpu.VMEM((CHUNK, 128), _f32),
            pltpu.VMEM((CHUNK, 128), _f32),
            pltpu.VMEM((CHUNK,), _i32),
            pltpu.VMEM((CHUNK,), _i32),
            pltpu.VMEM((CHUNK,), _i32),
            pltpu.VMEM_SHARED((Np, 128), _f32),
            pltpu.SemaphoreType.DMA,
        ],
    )
    def k(y_flat, src, dst, z128, acc_out,
          zbuf, rows, sidx, didx, gidx, acc_sh, sem):
        cid = lax.axis_index("c")
        sid = lax.axis_index("s")
        wid = _wid()
        pltpu.sync_copy(z128, zbuf)
        for cc in range(nc):
            _agg_pass(y_flat, src, dst, acc_sh, acc_out, zbuf, rows, sidx,
                      didx, gidx, sem, cc, nc, sid, cid, wid)

    return k


def _make_sc_gatherz():
    @functools.partial(
        pl.kernel,
        mesh=plsc.VectorSubcoreMesh(**_MESH),
        out_type=[
            jax.ShapeDtypeStruct((E, 128), _f32),
            jax.ShapeDtypeStruct((E, 128), _f32),
        ],
        scratch_types=[
            pltpu.VMEM((CHUNK, 128), _f32),
            pltpu.VMEM((CHUNK, 128), _f32),
            pltpu.VMEM((CHUNK,), _i32),
            pltpu.VMEM((CHUNK,), _i32),
            pltpu.SemaphoreType.DMA,
            pltpu.SemaphoreType.DMA,
        ],
    )
    def k(z_flat, src, dst, gs_out, gd_out, rows_s, rows_d, sidx, didx, sem_s, sem_d):
        wid = _wid()

        def body(i, carry):
            k_ = wid + i * 32

            @pl.when(k_ < NCH)
            def _():
                pltpu.sync_copy(src.at[pl.ds(k_ * CHUNK, CHUNK)], sidx)
                cp_s = pltpu.async_copy(z_flat.at[sidx], rows_s, sem_s)
                pltpu.sync_copy(dst.at[pl.ds(k_ * CHUNK, CHUNK)], didx)
                cp_d = pltpu.async_copy(z_flat.at[didx], rows_d, sem_d)
                cp_s.wait()
                pltpu.sync_copy(rows_s, gs_out.at[pl.ds(k_ * CHUNK, CHUNK), :])
                cp_d.wait()
                pltpu.sync_copy(rows_d, gd_out.at[pl.ds(k_ * CHUNK, CHUNK), :])

            return carry

        lax.fori_loop(0, ITERS, body, 0)

    return k


def _row_mask(nstep):
    rows = nstep * BNp + lax.broadcasted_iota(_i32, (BNp, 1), 0)
    return (rows < N).astype(_f32)


def _acc_stats(stats_ref, t, nstep, masked):
    tm = t * _row_mask(nstep) if masked else t
    s1 = jnp.sum(tm, axis=0, keepdims=True)
    s2 = jnp.sum(tm * tm, axis=0, keepdims=True)
    sts = jnp.concatenate([s1, s2], axis=0)

    @pl.when(nstep == 0)
    def _():
        stats_ref[...] = sts

    @pl.when(nstep > 0)
    def _():
        stats_ref[...] = stats_ref[...] + sts


def _make_k1a(nc_in, nc_out, di, do, with_stats):
    def body(acc, tp, aP, cP, deg, wl, wr, b, out, stats, u_s, hn_s):
        n = pl.program_id(0)
        d = deg[...]
        inv = 1.0 / jnp.maximum(d, 1.0)
        nzv = (d > 0.0).astype(_f32)
        for cc in range(nc_in):
            sl = slice(cc * 128, (cc + 1) * 128)
            a2 = acc[0, cc] + acc[1, cc]
            u_s[:, sl] = aP[:, sl] * (a2 * inv) + cP[:, sl] * nzv
            hn_s[:, sl] = aP[:, sl] * tp[cc] + cP[:, sl]
        pre = (jnp.dot(u_s[...], wl[...], preferred_element_type=_f32)
               + jnp.dot(hn_s[...], wr[...], preferred_element_type=_f32)
               + b[...])
        t = jnp.maximum(pre, 0.0)
        for co in range(nc_out):
            out[co] = t[:, co * 128:(co + 1) * 128]
        if with_stats:
            _acc_stats(stats, t, n, masked=True)

    out_shape = [jax.ShapeDtypeStruct((nc_out, Np, 128), _f32)]
    out_specs = [pl.BlockSpec((nc_out, BNp, 128), lambda n: (0, n, 0))]
    if with_stats:
        out_shape.append(jax.ShapeDtypeStruct((2, do), _f32))
        out_specs.append(pl.BlockSpec((2, do), lambda n: (0, 0)))
    else:
        def body_ns(acc, tp, aP, cP, deg, wl, wr, b, out, u_s, hn_s):
            body(acc, tp, aP, cP, deg, wl, wr, b, out, None, u_s, hn_s)

    return pl.pallas_call(
        body if with_stats else body_ns,
        grid=(NBLK,),
        in_specs=[
            pl.BlockSpec((2, nc_in, BNp, 128), lambda n: (0, 0, n, 0)),
            pl.BlockSpec((nc_in, BNp, 128), lambda n: (0, n, 0)),
            pl.BlockSpec((1, di), lambda n: (0, 0)),
            pl.BlockSpec((1, di), lambda n: (0, 0)),
            pl.BlockSpec((BNp, 1), lambda n: (n, 0)),
            pl.BlockSpec((di, do), lambda n: (0, 0)),
            pl.BlockSpec((di, do), lambda n: (0, 0)),
            pl.BlockSpec((1, do), lambda n: (0, 0)),
        ],
        out_specs=out_specs if with_stats else out_specs[0],
        out_shape=out_shape if with_stats else out_shape[0],
        scratch_shapes=[pltpu.VMEM((BNp, di), _f32), pltpu.VMEM((BNp, di), _f32)],
    )


def _make_k0b(nc_in, nc_out, di, do):
    def body(tp, aP, cP, wl, out, hn_s):
        for cc in range(nc_in):
            sl = slice(cc * 128, (cc + 1) * 128)
            hn_s[:, sl] = aP[:, sl] * tp[cc] + cP[:, sl]
        y = jnp.dot(hn_s[...], wl[...], preferred_element_type=_f32)
        for co in range(nc_out):
            out[co] = y[:, co * 128:(co + 1) * 128]

    return pl.pallas_call(
        body,
        grid=(NBLK,),
        in_specs=[
            pl.BlockSpec((nc_in, BNp, 128), lambda n: (0, n, 0)),
            pl.BlockSpec((1, di), lambda n: (0, 0)),
            pl.BlockSpec((1, di), lambda n: (0, 0)),
            pl.BlockSpec((di, do), lambda n: (0, 0)),
        ],
        out_specs=pl.BlockSpec((nc_out, BNp, 128), lambda n: (0, n, 0)),
        out_shape=jax.ShapeDtypeStruct((nc_out, Np, 128), _f32),
        scratch_shapes=[pltpu.VMEM((BNp, di), _f32)],
    )


def _make_k1b(nc_in, nc_out, di, do):
    def body(acc, tp, aP, cP, deg, wr, b, out, stats, u_s, hn_s):
        n = pl.program_id(0)
        d = deg[...]
        inv = 1.0 / jnp.maximum(d, 1.0)
        for co in range(nc_out):
            sl = slice(co * 128, (co + 1) * 128)
            u_s[:, sl] = (acc[0, co] + acc[1, co]) * inv
        for cc in range(nc_in):
            sl = slice(cc * 128, (cc + 1) * 128)
            hn_s[:, sl] = aP[:, sl] * tp[cc] + cP[:, sl]
        pre = (u_s[...]
               + jnp.dot(hn_s[...], wr[...], preferred_element_type=_f32)
               + b[...])
        t = jnp.maximum(pre, 0.0)
        for co in range(nc_out):
            out[co] = t[:, co * 128:(co + 1) * 128]
        _acc_stats(stats, t, n, masked=True)

    return pl.pallas_call(
        body,
        grid=(NBLK,),
        in_specs=[
            pl.BlockSpec((2, nc_out, BNp, 128), lambda n: (0, 0, n, 0)),
            pl.BlockSpec((nc_in, BNp, 128), lambda n: (0, n, 0)),
            pl.BlockSpec((1, di), lambda n: (0, 0)),
            pl.BlockSpec((1, di), lambda n: (0, 0)),
            pl.BlockSpec((BNp, 1), lambda n: (n, 0)),
            pl.BlockSpec((di, do), lambda n: (0, 0)),
            pl.BlockSpec((1, do), lambda n: (0, 0)),
        ],
        out_specs=[
            pl.BlockSpec((nc_out, BNp, 128), lambda n: (0, n, 0)),
            pl.BlockSpec((2, do), lambda n: (0, 0)),
        ],
        out_shape=[
            jax.ShapeDtypeStruct((nc_out, Np, 128), _f32),
            jax.ShapeDtypeStruct((2, do), _f32),
        ],
        scratch_shapes=[pltpu.VMEM((BNp, do), _f32), pltpu.VMEM((BNp, di), _f32)],
    )


def _make_kmu():
    def body(acc, tp, aP, cP, deg, wl78, wr78, b78, eps, mu_o, lv_o, z_o):
        d = deg[...]
        inv = 1.0 / jnp.maximum(d, 1.0)
        nzv = (d > 0.0).astype(_f32)
        u = aP[...] * (acc[0, 0] + acc[1, 0]) * inv + cP[...] * nzv
        hn = aP[...] * tp[0] + cP[...]
        muv = (jnp.dot(u, wl78[...], preferred_element_type=_f32)
               + jnp.dot(hn, wr78[...], preferred_element_type=_f32)
               + b78[...])
        mu = muv[:, :64]
        lv = muv[:, 64:]
        z = mu + eps[...] * jnp.exp(0.5 * lv)
        mu_o[...] = mu
        lv_o[...] = lv
        z_o[...] = jnp.concatenate([z, jnp.zeros_like(z)], axis=1)

    return pl.pallas_call(
        body,
        grid=(NBLK,),
        in_specs=[
            pl.BlockSpec((2, 1, BNp, 128), lambda n: (0, 0, n, 0)),
            pl.BlockSpec((1, BNp, 128), lambda n: (0, n, 0)),
            pl.BlockSpec((1, 128), lambda n: (0, 0)),
            pl.BlockSpec((1, 128), lambda n: (0, 0)),
            pl.BlockSpec((BNp, 1), lambda n: (n, 0)),
            pl.BlockSpec((128, 128), lambda n: (0, 0)),
            pl.BlockSpec((128, 128), lambda n: (0, 0)),
            pl.BlockSpec((1, 128), lambda n: (0, 0)),
            pl.BlockSpec((BNp, 64), lambda n: (n, 0)),
        ],
        out_specs=[
            pl.BlockSpec((BNp, 64), lambda n: (n, 0)),
            pl.BlockSpec((BNp, 64), lambda n: (n, 0)),
            pl.BlockSpec((BNp, 128), lambda n: (n, 0)),
        ],
        out_shape=[
            jax.ShapeDtypeStruct((Np, 64), _f32),
            jax.ShapeDtypeStruct((Np, 64), _f32),
            jax.ShapeDtypeStruct((Np, 128), _f32),
        ],
    )


def _make_dec1():
    def body(gs, gd, w, b, out, stats):
        n = pl.program_id(0)
        zd = gs[...] - gd[...]
        pre = jnp.dot(zd, w[...], preferred_element_type=_f32) + b[...]
        out[...] = pre
        _acc_stats(stats, pre, n, masked=False)

    return pl.pallas_call(
        body,
        grid=(GEB,),
        in_specs=[
            pl.BlockSpec((BE, 128), lambda n: (n, 0)),
            pl.BlockSpec((BE, 128), lambda n: (n, 0)),
            pl.BlockSpec((128, 128), lambda n: (0, 0)),
            pl.BlockSpec((1, 128), lambda n: (0, 0)),
        ],
        out_specs=[
            pl.BlockSpec((BE, 128), lambda n: (n, 0)),
            pl.BlockSpec((2, 128), lambda n: (0, 0)),
        ],
        out_shape=[
            jax.ShapeDtypeStruct((E, 128), _f32),
            jax.ShapeDtypeStruct((2, 128), _f32),
        ],
    )


def _make_dec_mid(di, do, with_stats):
    def body_s(xin, a, c, w, b, out, stats):
        n = pl.program_id(0)
        act = jnp.maximum(a[...] * xin[...] + c[...], 0.0)
        pre = jnp.dot(act, w[...], preferred_element_type=_f32) + b[...]
        out[...] = pre
        _acc_stats(stats, pre, n, masked=False)

    def body_ns(xin, a, c, w, b, out):
        act = jnp.maximum(a[...] * xin[...] + c[...], 0.0)
        out[...] = jnp.dot(act, w[...], preferred_element_type=_f32) + b[...]

    out_shape = [jax.ShapeDtypeStruct((E, do), _f32)]
    out_specs = [pl.BlockSpec((BE, do), lambda n: (n, 0))]
    if with_stats:
        out_shape.append(jax.ShapeDtypeStruct((2, do), _f32))
        out_specs.append(pl.BlockSpec((2, do), lambda n: (0, 0)))

    return pl.pallas_call(
        body_s if with_stats else body_ns,
        grid=(GEB,),
        in_specs=[
            pl.BlockSpec((BE, di), lambda n: (n, 0)),
            pl.BlockSpec((1, di), lambda n: (0, 0)),
            pl.BlockSpec((1, di), lambda n: (0, 0)),
            pl.BlockSpec((di, do), lambda n: (0, 0)),
            pl.BlockSpec((1, do), lambda n: (0, 0)),
        ],
        out_specs=out_specs if with_stats else out_specs[0],
        out_shape=out_shape if with_stats else out_shape[0],
    )


def _affine(stats, g, b, count):
    m = stats[0] / count
    v = stats[1] / count - m * m
    a = g * lax.rsqrt(v + 1e-5)
    c = b - a * m
    return a.reshape(1, -1), c.reshape(1, -1)


def kernel(x, edge_index, params):
    src = edge_index[0]
    dst = edge_index[1]
    convs = params["convs"]
    bns = params["bns"]
    decW = params["decW"]
    decb = params["decb"]
    dec_bn = params["dec_bn"]

    xp = jnp.pad(x, ((0, Np - N), (0, 0)))
    z128 = jnp.zeros((CHUNK, 128), _f32)
    ones128 = jnp.ones((CHUNK, 128), _f32)

    degp = _make_sc_deg()(dst, z128, ones128)
    deg = (degp[0, :, 0] + degp[1, :, 0]).reshape(Np, 1)
    acc0 = _make_sc_agg(1)(xp.reshape(1 * Np, 128), src, dst, z128)

    a_id = jnp.ones((1, 128), _f32)
    c_id = jnp.zeros((1, 128), _f32)

    def w2(i):
        cv = convs[i]
        return cv["Wl"], cv["Wr"], cv["b"].reshape(1, -1)

    t_prev = xp.reshape(1, Np, 128)
    aP, cP = a_id, c_id
    acc = acc0
    for i, (nc_in, nc_out) in enumerate([(1, 2), (2, 4), (4, 8), (8, 8)]):
        di, do = 128 * nc_in, 128 * nc_out
        wl, wr, b = w2(i)
        t_cur, st = _make_k1a(nc_in, nc_out, di, do, True)(
            acc, t_prev, aP, cP, deg, wl, wr, b)
        aP, cP = _affine(st, bns[i]["g"], bns[i]["b"], float(N))
        if i < 3:
            acc = _make_sc_agg(nc_out)(t_cur.reshape(nc_out * Np, 128), src,
                                       dst, z128)
        t_prev = t_cur

    for i, (nc_in, nc_out) in zip([4, 5, 6], [(8, 4), (4, 2), (2, 1)]):
        di, do = 128 * nc_in, 128 * nc_out
        wl, wr, b = w2(i)
        y = _make_k0b(nc_in, nc_out, di, do)(t_prev, aP, cP, wl)
        acc_y = _make_sc_agg(nc_out)(y.reshape(nc_out * Np, 128), src, dst, z128)
        t_cur, st = _make_k1b(nc_in, nc_out, di, do)(
            acc_y, t_prev, aP, cP, deg, wr, b)
        aP, cP = _affine(st, bns[i]["g"], bns[i]["b"], float(N))
        t_prev = t_cur

    acc6 = _make_sc_agg(1)(t_prev.reshape(Np, 128), src, dst, z128)
    wl78 = jnp.concatenate([convs[7]["Wl"], convs[8]["Wl"]], axis=1)
    wr78 = jnp.concatenate([convs[7]["Wr"], convs[8]["Wr"]], axis=1)
    b78 = jnp.concatenate([convs[7]["b"], convs[8]["b"]]).reshape(1, 128)
    eps = jax.random.normal(jax.random.key(42), (N, 64), _f32)
    eps_p = jnp.pad(eps, ((0, Np - N), (0, 0)))
    mu_p, lv_p, zpad = _make_kmu()(acc6, t_prev, aP, cP, deg, wl78, wr78,
                                   b78, eps_p)

    gs, gd = _make_sc_gatherz()(zpad, src, dst)
    w0p = jnp.concatenate([decW[0], jnp.zeros((64, 128), _f32)], axis=0)
    x0, ds0 = _make_dec1()(gs, gd, w0p, decb[0].reshape(1, -1))
    a0, c0 = _affine(ds0, dec_bn[0]["g"], dec_bn[0]["b"], float(E))
    x1, ds1 = _make_dec_mid(128, 128, True)(x0, a0, c0, decW[1],
                                            decb[1].reshape(1, -1))
    a1, c1 = _affine(ds1, dec_bn[1]["g"], dec_bn[1]["b"], float(E))
    x2, ds2 = _make_dec_mid(128, 64, True)(x1, a1, c1, decW[2],
                                           decb[2].reshape(1, -1))
    a2, c2 = _affine(ds2, dec_bn[2]["g"], dec_bn[2]["b"], float(E))
    w3p = jnp.concatenate([decW[3], jnp.zeros((64, 2), _f32)], axis=1)
    b3p = jnp.concatenate([decb[3], jnp.zeros((2,), _f32)]).reshape(1, 8)
    recon_p = _make_dec_mid(64, 8, False)(x2, a2, c2, w3p, b3p)

    return (recon_p[:, :6], mu_p[:N], lv_p[:N])


# speedup vs baseline: 5.0847x; 1.3028x over previous
import functools

import jax
import jax.numpy as jnp
from jax import lax
from jax.experimental import pallas as pl
from jax.experimental.pallas import tpu as pltpu
from jax.experimental.pallas import tpu_sc as plsc

N = 10000
E = 160000
Np = 10240
BNp = 1024
NBLK = Np // BNp
ROWS_PER_TILE = Np // 16
CHUNK = 128
NCH = E // CHUNK
ITERS = (NCH + 31) // 32
BE = 4000
GEB = E // BE

_f32 = jnp.float32
_i32 = jnp.int32

_MESH = dict(core_axis_name="c", subcore_axis_name="s")


def _wid():
    return lax.axis_index("s") * 2 + lax.axis_index("c")


ZROWS = 64


def _agg_pass(y_flat, src, dst, acc_sh, acc_out, zbuf, rows2, sidx2, didx2,
              gidx2, sem2, cc, nc, sid, cid, wid):
    for zz in range(ROWS_PER_TILE // ZROWS):
        pltpu.sync_copy(
            zbuf, acc_sh.at[pl.ds(sid * ROWS_PER_TILE + zz * ZROWS, ZROWS), :])
    plsc.subcore_barrier()

    def fire(b, k):
        @pl.when(k < NCH)
        def _():
            pltpu.sync_copy(src.at[pl.ds(k * CHUNK, CHUNK)], sidx2[b])
            if nc > 1:
                for j in range(CHUNK // 16):
                    gidx2[b][pl.ds(j * 16, 16)] = (
                        sidx2[b][pl.ds(j * 16, 16)] + _i32(cc * Np))
                idxref = gidx2[b]
            else:
                idxref = sidx2[b]
            pltpu.make_async_copy(y_flat.at[idxref], rows2[b], sem2[b]).start()

    def drain(b, k):
        @pl.when(k < NCH)
        def _():
            pltpu.sync_copy(dst.at[pl.ds(k * CHUNK, CHUNK)], didx2[b])
            pltpu.make_async_copy(y_flat.at[sidx2[b]], rows2[b], sem2[b]).wait()
            pltpu.sync_copy(rows2[b], acc_sh.at[didx2[b]], add=True)

    def body(g, carry):
        k0 = wid + (2 * g) * 32
        k1 = wid + (2 * g + 1) * 32
        fire(0, k0)
        fire(1, k1)
        drain(0, k0)
        drain(1, k1)
        return carry

    lax.fori_loop(0, ITERS // 2, body, 0)
    plsc.subcore_barrier()
    for zz in range(ROWS_PER_TILE // CHUNK):
        pltpu.sync_copy(
            acc_sh.at[pl.ds(sid * ROWS_PER_TILE + zz * CHUNK, CHUNK), :],
            rows2[0])
        pltpu.sync_copy(
            rows2[0],
            acc_out.at[cid, cc, pl.ds(sid * ROWS_PER_TILE + zz * CHUNK, CHUNK), :])
    plsc.subcore_barrier()


def _make_sc_deg():
    @functools.partial(
        pl.kernel,
        mesh=plsc.VectorSubcoreMesh(**_MESH),
        out_type=jax.ShapeDtypeStruct((2, Np, 128), _f32),
        scratch_types=[
            pltpu.VMEM((CHUNK,), _i32),
            pltpu.VMEM((CHUNK, 128), _f32),
            pltpu.VMEM((CHUNK, 128), _f32),
            pltpu.VMEM_SHARED((Np, 128), _f32),
        ],
    )
    def k(dst, z128, ones128, deg_out, didx, onesb, stg, deg_sh):
        cid = lax.axis_index("c")
        sid = lax.axis_index("s")
        wid = _wid()
        pltpu.sync_copy(ones128, onesb)
        pltpu.sync_copy(z128, stg)
        for zz in range(ROWS_PER_TILE // CHUNK):
            pltpu.sync_copy(
                stg, deg_sh.at[pl.ds(sid * ROWS_PER_TILE + zz * CHUNK, CHUNK), :])
        plsc.subcore_barrier()

        def dbody(i, carry):
            k_ = wid + i * 32

            @pl.when(k_ < NCH)
            def _():
                pltpu.sync_copy(dst.at[pl.ds(k_ * CHUNK, CHUNK)], didx)
                pltpu.sync_copy(onesb, deg_sh.at[didx], add=True)

            return carry

        lax.fori_loop(0, ITERS, dbody, 0)
        plsc.subcore_barrier()
        for zz in range(ROWS_PER_TILE // CHUNK):
            pltpu.sync_copy(
                deg_sh.at[pl.ds(sid * ROWS_PER_TILE + zz * CHUNK, CHUNK), :], stg)
            pltpu.sync_copy(
                stg,
                deg_out.at[cid, pl.ds(sid * ROWS_PER_TILE + zz * CHUNK, CHUNK), :])

    return k


def _make_sc_agg(nc):
    @functools.partial(
        pl.kernel,
        mesh=plsc.VectorSubcoreMesh(**_MESH),
        out_type=jax.ShapeDtypeStruct((2, nc, Np, 128), _f32),
        scratch_types=[
            pltpu.VMEM((ZROWS, 128), _f32),
            pltpu.VMEM((CHUNK, 128), _f32),
            pltpu.VMEM((CHUNK, 128), _f32),
            pltpu.VMEM((CHUNK,), _i32),
            pltpu.VMEM((CHUNK,), _i32),
            pltpu.VMEM((CHUNK,), _i32),
            pltpu.VMEM((CHUNK,), _i32),
            pltpu.VMEM((CHUNK,), _i32),
            pltpu.VMEM((CHUNK,), _i32),
            pltpu.VMEM_SHARED((Np, 128), _f32),
            pltpu.SemaphoreType.DMA,
            pltpu.SemaphoreType.DMA,
        ],
    )
    def k(y_flat, src, dst, z128, acc_out,
          zbuf, rows0, rows1, sidx0, sidx1, didx0, didx1, gidx0, gidx1,
          acc_sh, sem0, sem1):
        cid = lax.axis_index("c")
        sid = lax.axis_index("s")
        wid = _wid()
        pltpu.sync_copy(z128.at[pl.ds(0, ZROWS), :], zbuf)
        for cc in range(nc):
            _agg_pass(y_flat, src, dst, acc_sh, acc_out, zbuf,
                      (rows0, rows1), (sidx0, sidx1), (didx0, didx1),
                      (gidx0, gidx1), (sem0, sem1), cc, nc, sid, cid, wid)

    return k


def _make_sc_gatherz():
    @functools.partial(
        pl.kernel,
        mesh=plsc.VectorSubcoreMesh(**_MESH),
        out_type=[
            jax.ShapeDtypeStruct((E, 128), _f32),
            jax.ShapeDtypeStruct((E, 128), _f32),
        ],
        scratch_types=[
            pltpu.VMEM((CHUNK, 128), _f32),
            pltpu.VMEM((CHUNK, 128), _f32),
            pltpu.VMEM((CHUNK,), _i32),
            pltpu.VMEM((CHUNK,), _i32),
            pltpu.SemaphoreType.DMA,
            pltpu.SemaphoreType.DMA,
        ],
    )
    def k(z_flat, src, dst, gs_out, gd_out, rows_s, rows_d, sidx, didx, sem_s, sem_d):
        wid = _wid()

        def body(i, carry):
            k_ = wid + i * 32

            @pl.when(k_ < NCH)
            def _():
                pltpu.sync_copy(src.at[pl.ds(k_ * CHUNK, CHUNK)], sidx)
                cp_s = pltpu.async_copy(z_flat.at[sidx], rows_s, sem_s)
                pltpu.sync_copy(dst.at[pl.ds(k_ * CHUNK, CHUNK)], didx)
                cp_d = pltpu.async_copy(z_flat.at[didx], rows_d, sem_d)
                cp_s.wait()
                pltpu.sync_copy(rows_s, gs_out.at[pl.ds(k_ * CHUNK, CHUNK), :])
                cp_d.wait()
                pltpu.sync_copy(rows_d, gd_out.at[pl.ds(k_ * CHUNK, CHUNK), :])

            return carry

        lax.fori_loop(0, ITERS, body, 0)

    return k


def _row_mask(nstep):
    rows = nstep * BNp + lax.broadcasted_iota(_i32, (BNp, 1), 0)
    return (rows < N).astype(_f32)


def _acc_stats(stats_ref, t, nstep, masked):
    tm = t * _row_mask(nstep) if masked else t
    s1 = jnp.sum(tm, axis=0, keepdims=True)
    s2 = jnp.sum(tm * tm, axis=0, keepdims=True)
    sts = jnp.concatenate([s1, s2], axis=0)

    @pl.when(nstep == 0)
    def _():
        stats_ref[...] = sts

    @pl.when(nstep > 0)
    def _():
        stats_ref[...] = stats_ref[...] + sts


def _make_k1a(nc_in, nc_out, di, do, with_stats):
    def body(acc, tp, aP, cP, deg, wl, wr, b, out, stats, u_s, hn_s):
        n = pl.program_id(0)
        d = deg[...]
        inv = 1.0 / jnp.maximum(d, 1.0)
        nzv = (d > 0.0).astype(_f32)
        for cc in range(nc_in):
            sl = slice(cc * 128, (cc + 1) * 128)
            a2 = acc[0, cc] + acc[1, cc]
            u_s[:, sl] = aP[:, sl] * (a2 * inv) + cP[:, sl] * nzv
            hn_s[:, sl] = aP[:, sl] * tp[cc] + cP[:, sl]
        pre = (jnp.dot(u_s[...], wl[...], preferred_element_type=_f32)
               + jnp.dot(hn_s[...], wr[...], preferred_element_type=_f32)
               + b[...])
        t = jnp.maximum(pre, 0.0)
        for co in range(nc_out):
            out[co] = t[:, co * 128:(co + 1) * 128]
        if with_stats:
            _acc_stats(stats, t, n, masked=True)

    out_shape = [jax.ShapeDtypeStruct((nc_out, Np, 128), _f32)]
    out_specs = [pl.BlockSpec((nc_out, BNp, 128), lambda n: (0, n, 0))]
    if with_stats:
        out_shape.append(jax.ShapeDtypeStruct((2, do), _f32))
        out_specs.append(pl.BlockSpec((2, do), lambda n: (0, 0)))
    else:
        def body_ns(acc, tp, aP, cP, deg, wl, wr, b, out, u_s, hn_s):
            body(acc, tp, aP, cP, deg, wl, wr, b, out, None, u_s, hn_s)

    return pl.pallas_call(
        body if with_stats else body_ns,
        grid=(NBLK,),
        in_specs=[
            pl.BlockSpec((2, nc_in, BNp, 128), lambda n: (0, 0, n, 0)),
            pl.BlockSpec((nc_in, BNp, 128), lambda n: (0, n, 0)),
            pl.BlockSpec((1, di), lambda n: (0, 0)),
            pl.BlockSpec((1, di), lambda n: (0, 0)),
            pl.BlockSpec((BNp, 1), lambda n: (n, 0)),
            pl.BlockSpec((di, do), lambda n: (0, 0)),
            pl.BlockSpec((di, do), lambda n: (0, 0)),
            pl.BlockSpec((1, do), lambda n: (0, 0)),
        ],
        out_specs=out_specs if with_stats else out_specs[0],
        out_shape=out_shape if with_stats else out_shape[0],
        scratch_shapes=[pltpu.VMEM((BNp, di), _f32), pltpu.VMEM((BNp, di), _f32)],
    )


def _make_k0b(nc_in, nc_out, di, do):
    def body(tp, aP, cP, wl, out, hn_s):
        for cc in range(nc_in):
            sl = slice(cc * 128, (cc + 1) * 128)
            hn_s[:, sl] = aP[:, sl] * tp[cc] + cP[:, sl]
        y = jnp.dot(hn_s[...], wl[...], preferred_element_type=_f32)
        for co in range(nc_out):
            out[co] = y[:, co * 128:(co + 1) * 128]

    return pl.pallas_call(
        body,
        grid=(NBLK,),
        in_specs=[
            pl.BlockSpec((nc_in, BNp, 128), lambda n: (0, n, 0)),
            pl.BlockSpec((1, di), lambda n: (0, 0)),
            pl.BlockSpec((1, di), lambda n: (0, 0)),
            pl.BlockSpec((di, do), lambda n: (0, 0)),
        ],
        out_specs=pl.BlockSpec((nc_out, BNp, 128), lambda n: (0, n, 0)),
        out_shape=jax.ShapeDtypeStruct((nc_out, Np, 128), _f32),
        scratch_shapes=[pltpu.VMEM((BNp, di), _f32)],
    )


def _make_k1b(nc_in, nc_out, di, do):
    def body(acc, tp, aP, cP, deg, wr, b, out, stats, u_s, hn_s):
        n = pl.program_id(0)
        d = deg[...]
        inv = 1.0 / jnp.maximum(d, 1.0)
        for co in range(nc_out):
            sl = slice(co * 128, (co + 1) * 128)
            u_s[:, sl] = (acc[0, co] + acc[1, co]) * inv
        for cc in range(nc_in):
            sl = slice(cc * 128, (cc + 1) * 128)
            hn_s[:, sl] = aP[:, sl] * tp[cc] + cP[:, sl]
        pre = (u_s[...]
               + jnp.dot(hn_s[...], wr[...], preferred_element_type=_f32)
               + b[...])
        t = jnp.maximum(pre, 0.0)
        for co in range(nc_out):
            out[co] = t[:, co * 128:(co + 1) * 128]
        _acc_stats(stats, t, n, masked=True)

    return pl.pallas_call(
        body,
        grid=(NBLK,),
        in_specs=[
            pl.BlockSpec((2, nc_out, BNp, 128), lambda n: (0, 0, n, 0)),
            pl.BlockSpec((nc_in, BNp, 128), lambda n: (0, n, 0)),
            pl.BlockSpec((1, di), lambda n: (0, 0)),
            pl.BlockSpec((1, di), lambda n: (0, 0)),
            pl.BlockSpec((BNp, 1), lambda n: (n, 0)),
            pl.BlockSpec((di, do), lambda n: (0, 0)),
            pl.BlockSpec((1, do), lambda n: (0, 0)),
        ],
        out_specs=[
            pl.BlockSpec((nc_out, BNp, 128), lambda n: (0, n, 0)),
            pl.BlockSpec((2, do), lambda n: (0, 0)),
        ],
        out_shape=[
            jax.ShapeDtypeStruct((nc_out, Np, 128), _f32),
            jax.ShapeDtypeStruct((2, do), _f32),
        ],
        scratch_shapes=[pltpu.VMEM((BNp, do), _f32), pltpu.VMEM((BNp, di), _f32)],
    )


def _make_kmu():
    def body(acc, tp, aP, cP, deg, wl78, wr78, b78, eps, mu_o, lv_o, z_o):
        d = deg[...]
        inv = 1.0 / jnp.maximum(d, 1.0)
        nzv = (d > 0.0).astype(_f32)
        u = aP[...] * (acc[0, 0] + acc[1, 0]) * inv + cP[...] * nzv
        hn = aP[...] * tp[0] + cP[...]
        muv = (jnp.dot(u, wl78[...], preferred_element_type=_f32)
               + jnp.dot(hn, wr78[...], preferred_element_type=_f32)
               + b78[...])
        mu = muv[:, :64]
        lv = muv[:, 64:]
        z = mu + eps[...] * jnp.exp(0.5 * lv)
        mu_o[...] = mu
        lv_o[...] = lv
        z_o[...] = jnp.concatenate([z, jnp.zeros_like(z)], axis=1)

    return pl.pallas_call(
        body,
        grid=(NBLK,),
        in_specs=[
            pl.BlockSpec((2, 1, BNp, 128), lambda n: (0, 0, n, 0)),
            pl.BlockSpec((1, BNp, 128), lambda n: (0, n, 0)),
            pl.BlockSpec((1, 128), lambda n: (0, 0)),
            pl.BlockSpec((1, 128), lambda n: (0, 0)),
            pl.BlockSpec((BNp, 1), lambda n: (n, 0)),
            pl.BlockSpec((128, 128), lambda n: (0, 0)),
            pl.BlockSpec((128, 128), lambda n: (0, 0)),
            pl.BlockSpec((1, 128), lambda n: (0, 0)),
            pl.BlockSpec((BNp, 64), lambda n: (n, 0)),
        ],
        out_specs=[
            pl.BlockSpec((BNp, 64), lambda n: (n, 0)),
            pl.BlockSpec((BNp, 64), lambda n: (n, 0)),
            pl.BlockSpec((BNp, 128), lambda n: (n, 0)),
        ],
        out_shape=[
            jax.ShapeDtypeStruct((Np, 64), _f32),
            jax.ShapeDtypeStruct((Np, 64), _f32),
            jax.ShapeDtypeStruct((Np, 128), _f32),
        ],
    )


def _make_dec1():
    def body(gs, gd, w, b, out, stats):
        n = pl.program_id(0)
        zd = gs[...] - gd[...]
        pre = jnp.dot(zd, w[...], preferred_element_type=_f32) + b[...]
        out[...] = pre
        _acc_stats(stats, pre, n, masked=False)

    return pl.pallas_call(
        body,
        grid=(GEB,),
        in_specs=[
            pl.BlockSpec((BE, 128), lambda n: (n, 0)),
            pl.BlockSpec((BE, 128), lambda n: (n, 0)),
            pl.BlockSpec((128, 128), lambda n: (0, 0)),
            pl.BlockSpec((1, 128), lambda n: (0, 0)),
        ],
        out_specs=[
            pl.BlockSpec((BE, 128), lambda n: (n, 0)),
            pl.BlockSpec((2, 128), lambda n: (0, 0)),
        ],
        out_shape=[
            jax.ShapeDtypeStruct((E, 128), _f32),
            jax.ShapeDtypeStruct((2, 128), _f32),
        ],
    )


def _make_dec_mid(di, do, with_stats):
    def body_s(xin, a, c, w, b, out, stats):
        n = pl.program_id(0)
        act = jnp.maximum(a[...] * xin[...] + c[...], 0.0)
        pre = jnp.dot(act, w[...], preferred_element_type=_f32) + b[...]
        out[...] = pre
        _acc_stats(stats, pre, n, masked=False)

    def body_ns(xin, a, c, w, b, out):
        act = jnp.maximum(a[...] * xin[...] + c[...], 0.0)
        out[...] = jnp.dot(act, w[...], preferred_element_type=_f32) + b[...]

    out_shape = [jax.ShapeDtypeStruct((E, do), _f32)]
    out_specs = [pl.BlockSpec((BE, do), lambda n: (n, 0))]
    if with_stats:
        out_shape.append(jax.ShapeDtypeStruct((2, do), _f32))
        out_specs.append(pl.BlockSpec((2, do), lambda n: (0, 0)))

    return pl.pallas_call(
        body_s if with_stats else body_ns,
        grid=(GEB,),
        in_specs=[
            pl.BlockSpec((BE, di), lambda n: (n, 0)),
            pl.BlockSpec((1, di), lambda n: (0, 0)),
            pl.BlockSpec((1, di), lambda n: (0, 0)),
            pl.BlockSpec((di, do), lambda n: (0, 0)),
            pl.BlockSpec((1, do), lambda n: (0, 0)),
        ],
        out_specs=out_specs if with_stats else out_specs[0],
        out_shape=out_shape if with_stats else out_shape[0],
    )


def _affine(stats, g, b, count):
    m = stats[0] / count
    v = stats[1] / count - m * m
    a = g * lax.rsqrt(v + 1e-5)
    c = b - a * m
    return a.reshape(1, -1), c.reshape(1, -1)


def kernel(x, edge_index, params):
    src = edge_index[0]
    dst = edge_index[1]
    convs = params["convs"]
    bns = params["bns"]
    decW = params["decW"]
    decb = params["decb"]
    dec_bn = params["dec_bn"]

    xp = jnp.pad(x, ((0, Np - N), (0, 0)))
    z128 = jnp.zeros((CHUNK, 128), _f32)
    ones128 = jnp.ones((CHUNK, 128), _f32)

    degp = _make_sc_deg()(dst, z128, ones128)
    deg = (degp[0, :, 0] + degp[1, :, 0]).reshape(Np, 1)
    acc0 = _make_sc_agg(1)(xp.reshape(1 * Np, 128), src, dst, z128)

    a_id = jnp.ones((1, 128), _f32)
    c_id = jnp.zeros((1, 128), _f32)

    def w2(i):
        cv = convs[i]
        return cv["Wl"], cv["Wr"], cv["b"].reshape(1, -1)

    t_prev = xp.reshape(1, Np, 128)
    aP, cP = a_id, c_id
    acc = acc0
    for i, (nc_in, nc_out) in enumerate([(1, 2), (2, 4), (4, 8), (8, 8)]):
        di, do = 128 * nc_in, 128 * nc_out
        wl, wr, b = w2(i)
        t_cur, st = _make_k1a(nc_in, nc_out, di, do, True)(
            acc, t_prev, aP, cP, deg, wl, wr, b)
        aP, cP = _affine(st, bns[i]["g"], bns[i]["b"], float(N))
        if i < 3:
            acc = _make_sc_agg(nc_out)(t_cur.reshape(nc_out * Np, 128), src,
                                       dst, z128)
        t_prev = t_cur

    for i, (nc_in, nc_out) in zip([4, 5, 6], [(8, 4), (4, 2), (2, 1)]):
        di, do = 128 * nc_in, 128 * nc_out
        wl, wr, b = w2(i)
        y = _make_k0b(nc_in, nc_out, di, do)(t_prev, aP, cP, wl)
        acc_y = _make_sc_agg(nc_out)(y.reshape(nc_out * Np, 128), src, dst, z128)
        t_cur, st = _make_k1b(nc_in, nc_out, di, do)(
            acc_y, t_prev, aP, cP, deg, wr, b)
        aP, cP = _affine(st, bns[i]["g"], bns[i]["b"], float(N))
        t_prev = t_cur

    acc6 = _make_sc_agg(1)(t_prev.reshape(Np, 128), src, dst, z128)
    wl78 = jnp.concatenate([convs[7]["Wl"], convs[8]["Wl"]], axis=1)
    wr78 = jnp.concatenate([convs[7]["Wr"], convs[8]["Wr"]], axis=1)
    b78 = jnp.concatenate([convs[7]["b"], convs[8]["b"]]).reshape(1, 128)
    eps = jax.random.normal(jax.random.key(42), (N, 64), _f32)
    eps_p = jnp.pad(eps, ((0, Np - N), (0, 0)))
    mu_p, lv_p, zpad = _make_kmu()(acc6, t_prev, aP, cP, deg, wl78, wr78,
                                   b78, eps_p)

    gs, gd = _make_sc_gatherz()(zpad, src, dst)
    w0p = jnp.concatenate([decW[0], jnp.zeros((64, 128), _f32)], axis=0)
    x0, ds0 = _make_dec1()(gs, gd, w0p, decb[0].reshape(1, -1))
    a0, c0 = _affine(ds0, dec_bn[0]["g"], dec_bn[0]["b"], float(E))
    x1, ds1 = _make_dec_mid(128, 128, True)(x0, a0, c0, decW[1],
                                            decb[1].reshape(1, -1))
    a1, c1 = _affine(ds1, dec_bn[1]["g"], dec_bn[1]["b"], float(E))
    x2, ds2 = _make_dec_mid(128, 64, True)(x1, a1, c1, decW[2],
                                           decb[2].reshape(1, -1))
    a2, c2 = _affine(ds2, dec_bn[2]["g"], dec_bn[2]["b"], float(E))
    w3p = jnp.concatenate([decW[3], jnp.zeros((64, 2), _f32)], axis=1)
    b3p = jnp.concatenate([decb[3], jnp.zeros((2,), _f32)]).reshape(1, 8)
    recon_p = _make_dec_mid(64, 8, False)(x2, a2, c2, w3p, b3p)

    return (recon_p[:, :6], mu_p[:N], lv_p[:N])


# speedup vs baseline: 5.1828x; 1.0193x over previous
import functools

import jax
import jax.numpy as jnp
from jax import lax
from jax.experimental import pallas as pl
from jax.experimental.pallas import tpu as pltpu
from jax.experimental.pallas import tpu_sc as plsc

N = 10000
E = 160000
Np = 10240
BNp = 1024
NBLK = Np // BNp
ROWS_PER_TILE = Np // 16
CHUNK = 128
NCH = E // CHUNK
ITERS = (NCH + 31) // 32
BE = 4000
GEB = E // BE

_f32 = jnp.float32
_i32 = jnp.int32

_MESH = dict(core_axis_name="c", subcore_axis_name="s")


def _wid():
    return lax.axis_index("s") * 2 + lax.axis_index("c")


ZROWS = 64


def _agg_pass(y_flat, src, dst, acc_sh, acc_out, zbuf, rows2, sidx2, didx2,
              gidx2, sem2, cc, nc, sid, cid, wid):
    for zz in range(ROWS_PER_TILE // ZROWS):
        pltpu.sync_copy(
            zbuf, acc_sh.at[pl.ds(sid * ROWS_PER_TILE + zz * ZROWS, ZROWS), :])
    plsc.subcore_barrier()

    def fire(b, k):
        @pl.when(k < NCH)
        def _():
            pltpu.sync_copy(src.at[pl.ds(k * CHUNK, CHUNK)], sidx2[b])
            if nc > 1:
                for j in range(CHUNK // 16):
                    gidx2[b][pl.ds(j * 16, 16)] = (
                        sidx2[b][pl.ds(j * 16, 16)] + _i32(cc * Np))
                idxref = gidx2[b]
            else:
                idxref = sidx2[b]
            pltpu.make_async_copy(y_flat.at[idxref], rows2[b], sem2[b]).start()

    def drain(b, k):
        @pl.when(k < NCH)
        def _():
            pltpu.sync_copy(dst.at[pl.ds(k * CHUNK, CHUNK)], didx2[b])
            pltpu.make_async_copy(y_flat.at[sidx2[b]], rows2[b], sem2[b]).wait()
            pltpu.sync_copy(rows2[b], acc_sh.at[didx2[b]], add=True)

    def body(g, carry):
        k0 = wid + (2 * g) * 32
        k1 = wid + (2 * g + 1) * 32
        fire(0, k0)
        fire(1, k1)
        drain(0, k0)
        drain(1, k1)
        return carry

    lax.fori_loop(0, ITERS // 2, body, 0)
    plsc.subcore_barrier()
    for zz in range(ROWS_PER_TILE // CHUNK):
        pltpu.sync_copy(
            acc_sh.at[pl.ds(sid * ROWS_PER_TILE + zz * CHUNK, CHUNK), :],
            rows2[0])
        pltpu.sync_copy(
            rows2[0],
            acc_out.at[cid, cc, pl.ds(sid * ROWS_PER_TILE + zz * CHUNK, CHUNK), :])
    plsc.subcore_barrier()


def _make_sc_deg():
    @functools.partial(
        pl.kernel,
        mesh=plsc.VectorSubcoreMesh(**_MESH),
        out_type=jax.ShapeDtypeStruct((2, Np, 128), _f32),
        scratch_types=[
            pltpu.VMEM((CHUNK,), _i32),
            pltpu.VMEM((CHUNK, 128), _f32),
            pltpu.VMEM((CHUNK, 128), _f32),
            pltpu.VMEM_SHARED((Np, 128), _f32),
        ],
    )
    def k(dst, z128, ones128, deg_out, didx, onesb, stg, deg_sh):
        cid = lax.axis_index("c")
        sid = lax.axis_index("s")
        wid = _wid()
        pltpu.sync_copy(ones128, onesb)
        pltpu.sync_copy(z128, stg)
        for zz in range(ROWS_PER_TILE // CHUNK):
            pltpu.sync_copy(
                stg, deg_sh.at[pl.ds(sid * ROWS_PER_TILE + zz * CHUNK, CHUNK), :])
        plsc.subcore_barrier()

        def dbody(i, carry):
            k_ = wid + i * 32

            @pl.when(k_ < NCH)
            def _():
                pltpu.sync_copy(dst.at[pl.ds(k_ * CHUNK, CHUNK)], didx)
                pltpu.sync_copy(onesb, deg_sh.at[didx], add=True)

            return carry

        lax.fori_loop(0, ITERS, dbody, 0)
        plsc.subcore_barrier()
        for zz in range(ROWS_PER_TILE // CHUNK):
            pltpu.sync_copy(
                deg_sh.at[pl.ds(sid * ROWS_PER_TILE + zz * CHUNK, CHUNK), :], stg)
            pltpu.sync_copy(
                stg,
                deg_out.at[cid, pl.ds(sid * ROWS_PER_TILE + zz * CHUNK, CHUNK), :])

    return k


def _make_sc_agg(nc):
    @functools.partial(
        pl.kernel,
        mesh=plsc.VectorSubcoreMesh(**_MESH),
        out_type=jax.ShapeDtypeStruct((2, nc, Np, 128), _f32),
        scratch_types=[
            pltpu.VMEM((ZROWS, 128), _f32),
            pltpu.VMEM((CHUNK, 128), _f32),
            pltpu.VMEM((CHUNK, 128), _f32),
            pltpu.VMEM((CHUNK,), _i32),
            pltpu.VMEM((CHUNK,), _i32),
            pltpu.VMEM((CHUNK,), _i32),
            pltpu.VMEM((CHUNK,), _i32),
            pltpu.VMEM((CHUNK,), _i32),
            pltpu.VMEM((CHUNK,), _i32),
            pltpu.VMEM_SHARED((Np, 128), _f32),
            pltpu.SemaphoreType.DMA,
            pltpu.SemaphoreType.DMA,
        ],
    )
    def k(y_flat, src, dst, z128, acc_out,
          zbuf, rows0, rows1, sidx0, sidx1, didx0, didx1, gidx0, gidx1,
          acc_sh, sem0, sem1):
        cid = lax.axis_index("c")
        sid = lax.axis_index("s")
        wid = _wid()
        pltpu.sync_copy(z128.at[pl.ds(0, ZROWS), :], zbuf)
        for cc in range(nc):
            _agg_pass(y_flat, src, dst, acc_sh, acc_out, zbuf,
                      (rows0, rows1), (sidx0, sidx1), (didx0, didx1),
                      (gidx0, gidx1), (sem0, sem1), cc, nc, sid, cid, wid)

    return k


def _make_sc_gatherz():
    @functools.partial(
        pl.kernel,
        mesh=plsc.VectorSubcoreMesh(**_MESH),
        out_type=jax.ShapeDtypeStruct((E, 128), _f32),
        scratch_types=[
            pltpu.VMEM((CHUNK, 128), _f32),
            pltpu.VMEM((CHUNK, 128), _f32),
            pltpu.VMEM((CHUNK, 128), _f32),
            pltpu.VMEM((CHUNK, 128), _f32),
            pltpu.VMEM((CHUNK, 128), _f32),
            pltpu.VMEM((CHUNK, 128), _f32),
            pltpu.VMEM((CHUNK,), _i32),
            pltpu.VMEM((CHUNK,), _i32),
            pltpu.VMEM((CHUNK,), _i32),
            pltpu.VMEM((CHUNK,), _i32),
            pltpu.SemaphoreType.DMA,
            pltpu.SemaphoreType.DMA,
            pltpu.SemaphoreType.DMA,
            pltpu.SemaphoreType.DMA,
        ],
    )
    def k(z_flat, src, dst, z128, zd_out,
          rs0, rd0, rs1, rd1, zd0, zd1, sidx0, sidx1, didx0, didx1,
          ss0, sd0, ss1, sd1):
        wid = _wid()
        rs = (rs0, rs1)
        rd = (rd0, rd1)
        zd = (zd0, zd1)
        sidx = (sidx0, sidx1)
        didx = (didx0, didx1)
        ss = (ss0, ss1)
        sd = (sd0, sd1)
        pltpu.sync_copy(z128, zd0)
        pltpu.sync_copy(z128, zd1)

        def fire(b, k_):
            @pl.when(k_ < NCH)
            def _():
                pltpu.sync_copy(src.at[pl.ds(k_ * CHUNK, CHUNK)], sidx[b])
                pltpu.make_async_copy(z_flat.at[sidx[b]], rs[b], ss[b]).start()
                pltpu.sync_copy(dst.at[pl.ds(k_ * CHUNK, CHUNK)], didx[b])
                pltpu.make_async_copy(z_flat.at[didx[b]], rd[b], sd[b]).start()

        def drain(b, k_):
            @pl.when(k_ < NCH)
            def _():
                pltpu.make_async_copy(z_flat.at[sidx[b]], rs[b], ss[b]).wait()
                pltpu.make_async_copy(z_flat.at[didx[b]], rd[b], sd[b]).wait()

                def sub_row(r, carry):
                    for j in range(4):
                        zd[b][r, pl.ds(j * 16, 16)] = (
                            rs[b][r, pl.ds(j * 16, 16)]
                            - rd[b][r, pl.ds(j * 16, 16)])
                    return carry

                lax.fori_loop(0, CHUNK, sub_row, 0)
                pltpu.sync_copy(zd[b], zd_out.at[pl.ds(k_ * CHUNK, CHUNK), :])

        def body(g, carry):
            k0 = wid + (2 * g) * 32
            k1 = wid + (2 * g + 1) * 32
            fire(0, k0)
            fire(1, k1)
            drain(0, k0)
            drain(1, k1)
            return carry

        lax.fori_loop(0, ITERS // 2, body, 0)

    return k


def _row_mask(nstep):
    rows = nstep * BNp + lax.broadcasted_iota(_i32, (BNp, 1), 0)
    return (rows < N).astype(_f32)


def _acc_stats(stats_ref, t, nstep, masked):
    tm = t * _row_mask(nstep) if masked else t
    s1 = jnp.sum(tm, axis=0, keepdims=True)
    s2 = jnp.sum(tm * tm, axis=0, keepdims=True)
    sts = jnp.concatenate([s1, s2], axis=0)

    @pl.when(nstep == 0)
    def _():
        stats_ref[...] = sts

    @pl.when(nstep > 0)
    def _():
        stats_ref[...] = stats_ref[...] + sts


def _make_k1a(nc_in, nc_out, di, do, with_stats):
    def body(acc, tp, aP, cP, deg, wl, wr, b, out, stats, u_s, hn_s):
        n = pl.program_id(0)
        d = deg[...]
        inv = 1.0 / jnp.maximum(d, 1.0)
        nzv = (d > 0.0).astype(_f32)
        for cc in range(nc_in):
            sl = slice(cc * 128, (cc + 1) * 128)
            a2 = acc[0, cc] + acc[1, cc]
            u_s[:, sl] = aP[:, sl] * (a2 * inv) + cP[:, sl] * nzv
            hn_s[:, sl] = aP[:, sl] * tp[cc] + cP[:, sl]
        pre = (jnp.dot(u_s[...], wl[...], preferred_element_type=_f32)
               + jnp.dot(hn_s[...], wr[...], preferred_element_type=_f32)
               + b[...])
        t = jnp.maximum(pre, 0.0)
        for co in range(nc_out):
            out[co] = t[:, co * 128:(co + 1) * 128]
        if with_stats:
            _acc_stats(stats, t, n, masked=True)

    out_shape = [jax.ShapeDtypeStruct((nc_out, Np, 128), _f32)]
    out_specs = [pl.BlockSpec((nc_out, BNp, 128), lambda n: (0, n, 0))]
    if with_stats:
        out_shape.append(jax.ShapeDtypeStruct((2, do), _f32))
        out_specs.append(pl.BlockSpec((2, do), lambda n: (0, 0)))
    else:
        def body_ns(acc, tp, aP, cP, deg, wl, wr, b, out, u_s, hn_s):
            body(acc, tp, aP, cP, deg, wl, wr, b, out, None, u_s, hn_s)

    return pl.pallas_call(
        body if with_stats else body_ns,
        grid=(NBLK,),
        in_specs=[
            pl.BlockSpec((2, nc_in, BNp, 128), lambda n: (0, 0, n, 0)),
            pl.BlockSpec((nc_in, BNp, 128), lambda n: (0, n, 0)),
            pl.BlockSpec((1, di), lambda n: (0, 0)),
            pl.BlockSpec((1, di), lambda n: (0, 0)),
            pl.BlockSpec((BNp, 1), lambda n: (n, 0)),
            pl.BlockSpec((di, do), lambda n: (0, 0)),
            pl.BlockSpec((di, do), lambda n: (0, 0)),
            pl.BlockSpec((1, do), lambda n: (0, 0)),
        ],
        out_specs=out_specs if with_stats else out_specs[0],
        out_shape=out_shape if with_stats else out_shape[0],
        scratch_shapes=[pltpu.VMEM((BNp, di), _f32), pltpu.VMEM((BNp, di), _f32)],
    )


def _make_k0b(nc_in, nc_out, di, do):
    def body(tp, aP, cP, wl, out, hn_s):
        for cc in range(nc_in):
            sl = slice(cc * 128, (cc + 1) * 128)
            hn_s[:, sl] = aP[:, sl] * tp[cc] + cP[:, sl]
        y = jnp.dot(hn_s[...], wl[...], preferred_element_type=_f32)
        for co in range(nc_out):
            out[co] = y[:, co * 128:(co + 1) * 128]

    return pl.pallas_call(
        body,
        grid=(NBLK,),
        in_specs=[
            pl.BlockSpec((nc_in, BNp, 128), lambda n: (0, n, 0)),
            pl.BlockSpec((1, di), lambda n: (0, 0)),
            pl.BlockSpec((1, di), lambda n: (0, 0)),
            pl.BlockSpec((di, do), lambda n: (0, 0)),
        ],
        out_specs=pl.BlockSpec((nc_out, BNp, 128), lambda n: (0, n, 0)),
        out_shape=jax.ShapeDtypeStruct((nc_out, Np, 128), _f32),
        scratch_shapes=[pltpu.VMEM((BNp, di), _f32)],
    )


def _make_k1b(nc_in, nc_out, di, do):
    def body(acc, tp, aP, cP, deg, wr, b, out, stats, u_s, hn_s):
        n = pl.program_id(0)
        d = deg[...]
        inv = 1.0 / jnp.maximum(d, 1.0)
        for co in range(nc_out):
            sl = slice(co * 128, (co + 1) * 128)
            u_s[:, sl] = (acc[0, co] + acc[1, co]) * inv
        for cc in range(nc_in):
            sl = slice(cc * 128, (cc + 1) * 128)
            hn_s[:, sl] = aP[:, sl] * tp[cc] + cP[:, sl]
        pre = (u_s[...]
               + jnp.dot(hn_s[...], wr[...], preferred_element_type=_f32)
               + b[...])
        t = jnp.maximum(pre, 0.0)
        for co in range(nc_out):
            out[co] = t[:, co * 128:(co + 1) * 128]
        _acc_stats(stats, t, n, masked=True)

    return pl.pallas_call(
        body,
        grid=(NBLK,),
        in_specs=[
            pl.BlockSpec((2, nc_out, BNp, 128), lambda n: (0, 0, n, 0)),
            pl.BlockSpec((nc_in, BNp, 128), lambda n: (0, n, 0)),
            pl.BlockSpec((1, di), lambda n: (0, 0)),
            pl.BlockSpec((1, di), lambda n: (0, 0)),
            pl.BlockSpec((BNp, 1), lambda n: (n, 0)),
            pl.BlockSpec((di, do), lambda n: (0, 0)),
            pl.BlockSpec((1, do), lambda n: (0, 0)),
        ],
        out_specs=[
            pl.BlockSpec((nc_out, BNp, 128), lambda n: (0, n, 0)),
            pl.BlockSpec((2, do), lambda n: (0, 0)),
        ],
        out_shape=[
            jax.ShapeDtypeStruct((nc_out, Np, 128), _f32),
            jax.ShapeDtypeStruct((2, do), _f32),
        ],
        scratch_shapes=[pltpu.VMEM((BNp, do), _f32), pltpu.VMEM((BNp, di), _f32)],
    )


def _make_kmu():
    def body(acc, tp, aP, cP, deg, wl78, wr78, b78, eps, mu_o, lv_o, z_o):
        d = deg[...]
        inv = 1.0 / jnp.maximum(d, 1.0)
        nzv = (d > 0.0).astype(_f32)
        u = aP[...] * (acc[0, 0] + acc[1, 0]) * inv + cP[...] * nzv
        hn = aP[...] * tp[0] + cP[...]
        muv = (jnp.dot(u, wl78[...], preferred_element_type=_f32)
               + jnp.dot(hn, wr78[...], preferred_element_type=_f32)
               + b78[...])
        mu = muv[:, :64]
        lv = muv[:, 64:]
        z = mu + eps[...] * jnp.exp(0.5 * lv)
        mu_o[...] = mu
        lv_o[...] = lv
        z_o[...] = jnp.concatenate([z, jnp.zeros_like(z)], axis=1)

    return pl.pallas_call(
        body,
        grid=(NBLK,),
        in_specs=[
            pl.BlockSpec((2, 1, BNp, 128), lambda n: (0, 0, n, 0)),
            pl.BlockSpec((1, BNp, 128), lambda n: (0, n, 0)),
            pl.BlockSpec((1, 128), lambda n: (0, 0)),
            pl.BlockSpec((1, 128), lambda n: (0, 0)),
            pl.BlockSpec((BNp, 1), lambda n: (n, 0)),
            pl.BlockSpec((128, 128), lambda n: (0, 0)),
            pl.BlockSpec((128, 128), lambda n: (0, 0)),
            pl.BlockSpec((1, 128), lambda n: (0, 0)),
            pl.BlockSpec((BNp, 64), lambda n: (n, 0)),
        ],
        out_specs=[
            pl.BlockSpec((BNp, 64), lambda n: (n, 0)),
            pl.BlockSpec((BNp, 64), lambda n: (n, 0)),
            pl.BlockSpec((BNp, 128), lambda n: (n, 0)),
        ],
        out_shape=[
            jax.ShapeDtypeStruct((Np, 64), _f32),
            jax.ShapeDtypeStruct((Np, 64), _f32),
            jax.ShapeDtypeStruct((Np, 128), _f32),
        ],
    )


def _make_dec1():
    def body(zd, w, b, out, stats):
        n = pl.program_id(0)
        pre = jnp.dot(zd[...], w[...], preferred_element_type=_f32) + b[...]
        out[...] = pre
        _acc_stats(stats, pre, n, masked=False)

    return pl.pallas_call(
        body,
        grid=(GEB,),
        in_specs=[
            pl.BlockSpec((BE, 128), lambda n: (n, 0)),
            pl.BlockSpec((128, 128), lambda n: (0, 0)),
            pl.BlockSpec((1, 128), lambda n: (0, 0)),
        ],
        out_specs=[
            pl.BlockSpec((BE, 128), lambda n: (n, 0)),
            pl.BlockSpec((2, 128), lambda n: (0, 0)),
        ],
        out_shape=[
            jax.ShapeDtypeStruct((E, 128), _f32),
            jax.ShapeDtypeStruct((2, 128), _f32),
        ],
    )


def _make_dec_mid(di, do, with_stats):
    def body_s(xin, a, c, w, b, out, stats):
        n = pl.program_id(0)
        act = jnp.maximum(a[...] * xin[...] + c[...], 0.0)
        pre = jnp.dot(act, w[...], preferred_element_type=_f32) + b[...]
        out[...] = pre
        _acc_stats(stats, pre, n, masked=False)

    def body_ns(xin, a, c, w, b, out):
        act = jnp.maximum(a[...] * xin[...] + c[...], 0.0)
        out[...] = jnp.dot(act, w[...], preferred_element_type=_f32) + b[...]

    out_shape = [jax.ShapeDtypeStruct((E, do), _f32)]
    out_specs = [pl.BlockSpec((BE, do), lambda n: (n, 0))]
    if with_stats:
        out_shape.append(jax.ShapeDtypeStruct((2, do), _f32))
        out_specs.append(pl.BlockSpec((2, do), lambda n: (0, 0)))

    return pl.pallas_call(
        body_s if with_stats else body_ns,
        grid=(GEB,),
        in_specs=[
            pl.BlockSpec((BE, di), lambda n: (n, 0)),
            pl.BlockSpec((1, di), lambda n: (0, 0)),
            pl.BlockSpec((1, di), lambda n: (0, 0)),
            pl.BlockSpec((di, do), lambda n: (0, 0)),
            pl.BlockSpec((1, do), lambda n: (0, 0)),
        ],
        out_specs=out_specs if with_stats else out_specs[0],
        out_shape=out_shape if with_stats else out_shape[0],
    )


def _affine(stats, g, b, count):
    m = stats[0] / count
    v = stats[1] / count - m * m
    a = g * lax.rsqrt(v + 1e-5)
    c = b - a * m
    return a.reshape(1, -1), c.reshape(1, -1)


def kernel(x, edge_index, params):
    src = edge_index[0]
    dst = edge_index[1]
    convs = params["convs"]
    bns = params["bns"]
    decW = params["decW"]
    decb = params["decb"]
    dec_bn = params["dec_bn"]

    xp = jnp.pad(x, ((0, Np - N), (0, 0)))
    z128 = jnp.zeros((CHUNK, 128), _f32)
    ones128 = jnp.ones((CHUNK, 128), _f32)

    degp = _make_sc_deg()(dst, z128, ones128)
    deg = (degp[0, :, 0] + degp[1, :, 0]).reshape(Np, 1)
    acc0 = _make_sc_agg(1)(xp.reshape(1 * Np, 128), src, dst, z128)

    a_id = jnp.ones((1, 128), _f32)
    c_id = jnp.zeros((1, 128), _f32)

    def w2(i):
        cv = convs[i]
        return cv["Wl"], cv["Wr"], cv["b"].reshape(1, -1)

    t_prev = xp.reshape(1, Np, 128)
    aP, cP = a_id, c_id
    acc = acc0
    for i, (nc_in, nc_out) in enumerate([(1, 2), (2, 4), (4, 8), (8, 8)]):
        di, do = 128 * nc_in, 128 * nc_out
        wl, wr, b = w2(i)
        t_cur, st = _make_k1a(nc_in, nc_out, di, do, True)(
            acc, t_prev, aP, cP, deg, wl, wr, b)
        aP, cP = _affine(st, bns[i]["g"], bns[i]["b"], float(N))
        if i < 3:
            acc = _make_sc_agg(nc_out)(t_cur.reshape(nc_out * Np, 128), src,
                                       dst, z128)
        t_prev = t_cur

    for i, (nc_in, nc_out) in zip([4, 5, 6], [(8, 4), (4, 2), (2, 1)]):
        di, do = 128 * nc_in, 128 * nc_out
        wl, wr, b = w2(i)
        y = _make_k0b(nc_in, nc_out, di, do)(t_prev, aP, cP, wl)
        acc_y = _make_sc_agg(nc_out)(y.reshape(nc_out * Np, 128), src, dst, z128)
        t_cur, st = _make_k1b(nc_in, nc_out, di, do)(
            acc_y, t_prev, aP, cP, deg, wr, b)
        aP, cP = _affine(st, bns[i]["g"], bns[i]["b"], float(N))
        t_prev = t_cur

    acc6 = _make_sc_agg(1)(t_prev.reshape(Np, 128), src, dst, z128)
    wl78 = jnp.concatenate([convs[7]["Wl"], convs[8]["Wl"]], axis=1)
    wr78 = jnp.concatenate([convs[7]["Wr"], convs[8]["Wr"]], axis=1)
    b78 = jnp.concatenate([convs[7]["b"], convs[8]["b"]]).reshape(1, 128)
    eps = jax.random.normal(jax.random.key(42), (N, 64), _f32)
    eps_p = jnp.pad(eps, ((0, Np - N), (0, 0)))
    mu_p, lv_p, zpad = _make_kmu()(acc6, t_prev, aP, cP, deg, wl78, wr78,
                                   b78, eps_p)

    zd = _make_sc_gatherz()(zpad, src, dst, z128)
    w0p = jnp.concatenate([decW[0], jnp.zeros((64, 128), _f32)], axis=0)
    x0, ds0 = _make_dec1()(zd, w0p, decb[0].reshape(1, -1))
    a0, c0 = _affine(ds0, dec_bn[0]["g"], dec_bn[0]["b"], float(E))
    x1, ds1 = _make_dec_mid(128, 128, True)(x0, a0, c0, decW[1],
                                            decb[1].reshape(1, -1))
    a1, c1 = _affine(ds1, dec_bn[1]["g"], dec_bn[1]["b"], float(E))
    x2, ds2 = _make_dec_mid(128, 64, True)(x1, a1, c1, decW[2],
                                           decb[2].reshape(1, -1))
    a2, c2 = _affine(ds2, dec_bn[2]["g"], dec_bn[2]["b"], float(E))
    w3p = jnp.concatenate([decW[3], jnp.zeros((64, 2), _f32)], axis=1)
    b3p = jnp.concatenate([decb[3], jnp.zeros((2,), _f32)]).reshape(1, 8)
    recon_p = _make_dec_mid(64, 8, False)(x2, a2, c2, w3p, b3p)

    return (recon_p[:, :6], mu_p[:N], lv_p[:N])
